# Initial kernel scaffold; baseline (speedup 1.0000x reference)
import jax
import jax.numpy as jnp
from jax.experimental import pallas as pl


def kernel(node_features, edge_index, W_proj, b_proj, gcn1_W, gcn1_b, gcn2_W, gcn2_b, in_proj_W, conv_w, conv_b, x_proj_W, dt_proj_W, dt_proj_b, A_log, D_param, out_proj_W, ln_g, ln_b, critic_W, critic_b, hl_W, hl_b, mlt_W, mlt_b, p1_W, p1_b, p2_W, p2_b, dep_W, dep_b, sel_W, sel_b):
    raise NotImplementedError("write your pallas kernel here")


# speedup vs baseline: 60.6697x; 60.6697x over previous
import functools

import jax
import jax.numpy as jnp
from jax import lax
from jax.experimental import pallas as pl
from jax.experimental.pallas import tpu as pltpu
from jax.experimental.pallas import tpu_sc as plsc

N = 10000
E = 320000
F = 128
H = 128
DI = 256
DS = 16
DC = 4
DTR = 8
NS = 10000
NLC = 8

NC = 2
NSUB = 16
NW = NC * NSUB
EB = 125
ER = E // EB
RPW = ER // NW
RPS = N // NSUB

BLK = 1000
GN = N // BLK
CH = 200
GC = N // CH


def _sc_mesh():
    return plsc.VectorSubcoreMesh(
        core_axis_name="c", subcore_axis_name="s",
        num_cores=NC, num_subcores=NSUB)


NFC = N // 128
NTAIL = N - NFC * 128
ZB = 120
NZC = N // ZB
ZTAIL = N - NZC * ZB


def _sc_degree(dst2d):
    @functools.partial(
        pl.kernel,
        out_type=jax.ShapeDtypeStruct((NC, N, 16), jnp.float32),
        mesh=_sc_mesh(),
        scratch_types=[
            pltpu.VMEM((RPW, EB), jnp.int32),
            pltpu.VMEM((EB, 16), jnp.float32),
            pltpu.VMEM((128, 16), jnp.float32),
            pltpu.VMEM_SHARED((N, 16), jnp.float32),
        ],
    )
    def k(dst_hbm, out_hbm, didx, ones_v, zbuf, hist_sh):
        c = lax.axis_index("c")
        s = lax.axis_index("s")
        w = s * NC + c

        def fill_ones(i, _):
            ones_v[i, :] = jnp.ones((16,), jnp.float32)
            return 0

        lax.fori_loop(0, EB, fill_ones, 0)

        def fill_zero(i, _):
            zbuf[i, :] = jnp.zeros((16,), jnp.float32)
            return 0

        lax.fori_loop(0, 128, fill_zero, 0)

        def zero_slab(kk, _):
            i = s + NSUB * kk

            @pl.when(i < NFC)
            def _():
                pltpu.sync_copy(zbuf, hist_sh.at[pl.ds(i * 128, 128)])

            return 0

        lax.fori_loop(0, NFC // NSUB + 1, zero_slab, 0)

        @pl.when(s == 0)
        def _():
            pltpu.sync_copy(zbuf.at[pl.ds(0, NTAIL)],
                            hist_sh.at[pl.ds(NFC * 128, NTAIL)])

        plsc.subcore_barrier()

        pltpu.sync_copy(dst_hbm.at[pl.ds(w * RPW, RPW)], didx)

        def body(j, _):
            pltpu.sync_copy(ones_v, hist_sh.at[didx.at[j]], add=True)
            return 0

        lax.fori_loop(0, RPW, body, 0)
        plsc.subcore_barrier()

        def out_slab(kk, _):
            i = s + NSUB * kk

            @pl.when(i < NFC)
            def _():
                pltpu.sync_copy(hist_sh.at[pl.ds(i * 128, 128)], zbuf)
                pltpu.sync_copy(zbuf, out_hbm.at[c, pl.ds(i * 128, 128)])

            return 0

        lax.fori_loop(0, NFC // NSUB + 1, out_slab, 0)

        @pl.when(s == 0)
        def _():
            pltpu.sync_copy(hist_sh.at[pl.ds(NFC * 128, NTAIL)],
                            zbuf.at[pl.ds(0, NTAIL)])
            pltpu.sync_copy(zbuf.at[pl.ds(0, NTAIL)],
                            out_hbm.at[c, pl.ds(NFC * 128, NTAIL)])

    return k(dst2d)


def _sc_scatter(u, src2d, dst2d):
    @functools.partial(
        pl.kernel,
        out_type=jax.ShapeDtypeStruct((NC, N, H), jnp.float32),
        mesh=_sc_mesh(),
        scratch_types=[
            pltpu.VMEM((RPW, EB), jnp.int32),
            pltpu.VMEM((RPW, EB), jnp.int32),
            pltpu.VMEM((EB, H), jnp.float32),
            pltpu.VMEM_SHARED((N, H), jnp.float32),
            pltpu.SemaphoreType.DMA,
        ],
    )
    def k(u_hbm, src_hbm, dst_hbm, out_hbm, sidx, didx, rows, acc_sh, sem):
        c = lax.axis_index("c")
        s = lax.axis_index("s")
        w = s * NC + c

        def zero_row(i, _):
            def zero_chunk(kk, _2):
                rows[i, pl.ds(kk * 16, 16)] = jnp.zeros((16,), jnp.float32)
                return 0

            lax.fori_loop(0, H // 16, zero_chunk, 0)
            return 0

        lax.fori_loop(0, EB, zero_row, 0)

        def zero_slab(kk, _):
            i = s + NSUB * kk

            @pl.when(i < NZC)
            def _():
                pltpu.sync_copy(rows.at[pl.ds(0, ZB)],
                                acc_sh.at[pl.ds(i * ZB, ZB)])

            return 0

        lax.fori_loop(0, NZC // NSUB + 1, zero_slab, 0)

        @pl.when(s == 0)
        def _():
            pltpu.sync_copy(rows.at[pl.ds(0, ZTAIL)],
                            acc_sh.at[pl.ds(NZC * ZB, ZTAIL)])

        plsc.subcore_barrier()

        pltpu.sync_copy(src_hbm.at[pl.ds(w * RPW, RPW)], sidx)
        pltpu.sync_copy(dst_hbm.at[pl.ds(w * RPW, RPW)], didx)

        def body(j, _):
            pltpu.async_copy(u_hbm.at[sidx.at[j]], rows, sem).wait()
            pltpu.sync_copy(rows, acc_sh.at[didx.at[j]], add=True)
            return 0

        lax.fori_loop(0, RPW, body, 0)
        plsc.subcore_barrier()

        def out_slab(kk, _):
            i = s + NSUB * kk

            @pl.when(i < NZC)
            def _():
                pltpu.sync_copy(acc_sh.at[pl.ds(i * ZB, ZB)],
                                rows.at[pl.ds(0, ZB)])
                pltpu.sync_copy(rows.at[pl.ds(0, ZB)],
                                out_hbm.at[c, pl.ds(i * ZB, ZB)])

            return 0

        lax.fori_loop(0, NZC // NSUB + 1, out_slab, 0)

        @pl.when(s == 0)
        def _():
            pltpu.sync_copy(acc_sh.at[pl.ds(NZC * ZB, ZTAIL)],
                            rows.at[pl.ds(0, ZTAIL)])
            pltpu.sync_copy(rows.at[pl.ds(0, ZTAIL)],
                            out_hbm.at[c, pl.ds(NZC * ZB, ZTAIL)])

    return k(u, src2d, dst2d)


def _tc_pre1(x, histparts, Wp, bp, W1):
    def body(x_ref, hp_ref, wp_ref, bp_ref, w1_ref, u1_ref, dinv_ref):
        p = hp_ref[...]
        deg = p[0, :, 0:1] + p[1, :, 0:1] + 1.0
        dinv = lax.rsqrt(deg)
        h0 = jnp.dot(x_ref[...], wp_ref[...],
                     preferred_element_type=jnp.float32) + bp_ref[...]
        h0 = jnp.maximum(h0, 0.0)
        t1 = jnp.dot(h0, w1_ref[...], preferred_element_type=jnp.float32)
        u1_ref[...] = t1 * dinv
        dinv_ref[...] = dinv

    return pl.pallas_call(
        body,
        grid=(GN,),
        in_specs=[
            pl.BlockSpec((BLK, F), lambda i: (i, 0)),
            pl.BlockSpec((NC, BLK, 16), lambda i: (0, i, 0)),
            pl.BlockSpec((F, H), lambda i: (0, 0)),
            pl.BlockSpec((1, H), lambda i: (0, 0)),
            pl.BlockSpec((H, H), lambda i: (0, 0)),
        ],
        out_specs=[
            pl.BlockSpec((BLK, H), lambda i: (i, 0)),
            pl.BlockSpec((BLK, 1), lambda i: (i, 0)),
        ],
        out_shape=[
            jax.ShapeDtypeStruct((N, H), jnp.float32),
            jax.ShapeDtypeStruct((N, 1), jnp.float32),
        ],
    )(x, histparts, Wp, bp, W1)


def _tc_mid(accparts, u1, dinv, b1, W2):
    def body(p_ref, u1_ref, dinv_ref, b1_ref, w2_ref, u2_ref):
        p = p_ref[...]
        dinv = dinv_ref[...]
        g1 = dinv * (p[0] + p[1] + u1_ref[...]) + b1_ref[...]
        g1 = jnp.maximum(g1, 0.0)
        u2_ref[...] = jnp.dot(
            g1, w2_ref[...], preferred_element_type=jnp.float32) * dinv

    return pl.pallas_call(
        body,
        grid=(GN,),
        in_specs=[
            pl.BlockSpec((NC, BLK, H), lambda i: (0, i, 0)),
            pl.BlockSpec((BLK, H), lambda i: (i, 0)),
            pl.BlockSpec((BLK, 1), lambda i: (i, 0)),
            pl.BlockSpec((1, H), lambda i: (0, 0)),
            pl.BlockSpec((H, H), lambda i: (0, 0)),
        ],
        out_specs=pl.BlockSpec((BLK, H), lambda i: (i, 0)),
        out_shape=jax.ShapeDtypeStruct((N, H), jnp.float32),
    )(accparts, u1, dinv, b1, W2)


def _tc_post2(accparts, u2, dinv, b2, Win):
    def body(p_ref, u2_ref, dinv_ref, b2_ref, win_ref, hg_ref, xi_ref, zs_ref):
        p = p_ref[...]
        hg = dinv_ref[...] * (p[0] + p[1] + u2_ref[...]) + b2_ref[...]
        hg = jnp.maximum(hg, 0.0)
        hg_ref[...] = hg
        xz = jnp.dot(hg, win_ref[...], preferred_element_type=jnp.float32)
        xi_ref[...] = xz[:, :DI]
        z = xz[:, DI:]
        zs_ref[...] = z * jax.nn.sigmoid(z)

    return pl.pallas_call(
        body,
        grid=(GN,),
        in_specs=[
            pl.BlockSpec((NC, BLK, H), lambda i: (0, i, 0)),
            pl.BlockSpec((BLK, H), lambda i: (i, 0)),
            pl.BlockSpec((BLK, 1), lambda i: (i, 0)),
            pl.BlockSpec((1, H), lambda i: (0, 0)),
            pl.BlockSpec((H, 2 * DI), lambda i: (0, 0)),
        ],
        out_specs=[
            pl.BlockSpec((BLK, H), lambda i: (i, 0)),
            pl.BlockSpec((BLK, DI), lambda i: (i, 0)),
            pl.BlockSpec((BLK, DI), lambda i: (i, 0)),
        ],
        out_shape=[
            jax.ShapeDtypeStruct((N, H), jnp.float32),
            jax.ShapeDtypeStruct((N, DI), jnp.float32),
            jax.ShapeDtypeStruct((N, DI), jnp.float32),
        ],
    )(accparts, u2, dinv, b2, Win)


def _tc_conv(xi, convwT, cb, xpW, dtW, dtb):
    def body(cur_ref, prev_ref, cwT_ref, cb_ref, xpw_ref, dtw_ref, dtb_ref,
             xc_ref, dt_ref, bm_ref, cm_ref):
        pi = pl.program_id(0)
        prev_t = prev_ref[BLK - (DC - 1):BLK, :]
        prev_t = jnp.where(pi == 0, 0.0, prev_t)
        win = jnp.concatenate([prev_t, cur_ref[...]], axis=0)
        cw = cwT_ref[...]
        xc = win[DC - 1:DC - 1 + BLK] * cw[DC - 1:DC, :]
        for kk in range(DC - 1):
            xc = xc + win[kk:kk + BLK] * cw[kk:kk + 1, :]
        xc = xc + cb_ref[...]
        xc = xc * jax.nn.sigmoid(xc)
        xc_ref[...] = xc
        dbl = jnp.dot(xc, xpw_ref[...], preferred_element_type=jnp.float32)
        dtpre = jnp.dot(dbl[:, :DTR], dtw_ref[...],
                        preferred_element_type=jnp.float32) + dtb_ref[...]
        dt_ref[...] = jax.nn.softplus(dtpre)
        bm_ref[...] = dbl[:, DTR:DTR + DS]
        cm_ref[...] = dbl[:, DTR + DS:DTR + 2 * DS]

    return pl.pallas_call(
        body,
        grid=(GN,),
        in_specs=[
            pl.BlockSpec((BLK, DI), lambda i: (i, 0)),
            pl.BlockSpec((BLK, DI), lambda i: (jnp.maximum(i - 1, 0), 0)),
            pl.BlockSpec((DC, DI), lambda i: (0, 0)),
            pl.BlockSpec((1, DI), lambda i: (0, 0)),
            pl.BlockSpec((DI, DTR + 2 * DS), lambda i: (0, 0)),
            pl.BlockSpec((DTR, DI), lambda i: (0, 0)),
            pl.BlockSpec((1, DI), lambda i: (0, 0)),
        ],
        out_specs=[
            pl.BlockSpec((BLK, DI), lambda i: (i, 0)),
            pl.BlockSpec((BLK, DI), lambda i: (i, 0)),
            pl.BlockSpec((BLK, DS), lambda i: (i, 0)),
            pl.BlockSpec((BLK, DS), lambda i: (i, 0)),
        ],
        out_shape=[
            jax.ShapeDtypeStruct((N, DI), jnp.float32),
            jax.ShapeDtypeStruct((N, DI), jnp.float32),
            jax.ShapeDtypeStruct((N, DS), jnp.float32),
            jax.ShapeDtypeStruct((N, DS), jnp.float32),
        ],
    )(xi, xi, convwT, cb, xpW, dtW, dtb)


def _tc_scan(dt, xc, Bm, Cm, zs, hg, ATlog, D2, Wout, lng, lnb):
    def body(dt_ref, xc_ref, bm_ref, cm_ref, zs_ref, hg_ref, atl_ref, d_ref,
             wout_ref, lng_ref, lnb_ref, g_ref, dA_s, U_s, hst_s, h_s, gacc_s):
        pi = pl.program_id(0)
        AT = -jnp.exp(atl_ref[...])
        dt_b = dt_ref[...]
        xc_b = xc_ref[...]
        dA_s[...] = jnp.exp(dt_b[:, None, :] * AT[None])
        U_s[...] = bm_ref[...][:, :, None] * (dt_b * xc_b)[:, None, :]

        @pl.when(pi == 0)
        def _():
            h_s[...] = jnp.zeros((DS, DI), jnp.float32)
            gacc_s[...] = jnp.zeros((1, H), jnp.float32)

        def step(t, h):
            h2 = dA_s[t] * h + U_s[t]
            hst_s[t] = h2
            return h2

        h = lax.fori_loop(0, CH, step, h_s[...])
        h_s[...] = h

        Y = jnp.sum(hst_s[...] * cm_ref[...][:, :, None], axis=1)
        y = (Y + xc_b * d_ref[...]) * zs_ref[...]
        hm = jnp.dot(y, wout_ref[...], preferred_element_type=jnp.float32)
        hf = hg_ref[...] + hm
        mu = jnp.mean(hf, axis=-1, keepdims=True)
        var = jnp.mean((hf - mu) ** 2, axis=-1, keepdims=True)
        hf = (hf - mu) * lax.rsqrt(var + 1e-5) * lng_ref[...] + lnb_ref[...]
        gacc_s[...] += jnp.sum(hf, axis=0, keepdims=True)
        g_ref[...] = gacc_s[...] * (1.0 / N)

    return pl.pallas_call(
        body,
        grid=(GC,),
        in_specs=[
            pl.BlockSpec((CH, DI), lambda i: (i, 0)),
            pl.BlockSpec((CH, DI), lambda i: (i, 0)),
            pl.BlockSpec((CH, DS), lambda i: (i, 0)),
            pl.BlockSpec((CH, DS), lambda i: (i, 0)),
            pl.BlockSpec((CH, DI), lambda i: (i, 0)),
            pl.BlockSpec((CH, H), lambda i: (i, 0)),
            pl.BlockSpec((DS, DI), lambda i: (0, 0)),
            pl.BlockSpec((1, DI), lambda i: (0, 0)),
            pl.BlockSpec((DI, H), lambda i: (0, 0)),
            pl.BlockSpec((1, H), lambda i: (0, 0)),
            pl.BlockSpec((1, H), lambda i: (0, 0)),
        ],
        out_specs=pl.BlockSpec((1, H), lambda i: (0, 0)),
        out_shape=jax.ShapeDtypeStruct((1, H), jnp.float32),
        scratch_shapes=[
            pltpu.VMEM((CH, DS, DI), jnp.float32),
            pltpu.VMEM((CH, DS, DI), jnp.float32),
            pltpu.VMEM((CH, DS, DI), jnp.float32),
            pltpu.VMEM((DS, DI), jnp.float32),
            pltpu.VMEM((1, H), jnp.float32),
        ],
    )(dt, xc, Bm, Cm, zs, hg, ATlog, D2, Wout, lng, lnb)


def _tc_heads(g, ws, bs):
    nheads = len(ws)

    def body(*refs):
        g_v = refs[0][...]
        for j in range(nheads):
            w_ref = refs[1 + j]
            b_ref = refs[1 + nheads + j]
            o_ref = refs[1 + 2 * nheads + j]
            o_ref[...] = jnp.dot(
                g_v, w_ref[...], preferred_element_type=jnp.float32
            ) + b_ref[...]

    return pl.pallas_call(
        body,
        out_shape=[jax.ShapeDtypeStruct((1, w.shape[1]), jnp.float32)
                   for w in ws],
    )(g, *ws, *bs)


def kernel(node_features, edge_index, W_proj, b_proj, gcn1_W, gcn1_b, gcn2_W,
           gcn2_b, in_proj_W, conv_w, conv_b, x_proj_W, dt_proj_W, dt_proj_b,
           A_log, D_param, out_proj_W, ln_g, ln_b, critic_W, critic_b, hl_W,
           hl_b, mlt_W, mlt_b, p1_W, p1_b, p2_W, p2_b, dep_W, dep_b, sel_W,
           sel_b):
    src2d = edge_index[0].astype(jnp.int32).reshape(ER, EB)
    dst2d = edge_index[1].astype(jnp.int32).reshape(ER, EB)

    histparts = _sc_degree(dst2d)
    u1, dinv = _tc_pre1(node_features, histparts, W_proj,
                        b_proj.reshape(1, H), gcn1_W)
    acc1 = _sc_scatter(u1, src2d, dst2d)
    u2 = _tc_mid(acc1, u1, dinv, gcn1_b.reshape(1, H), gcn2_W)
    acc2 = _sc_scatter(u2, src2d, dst2d)
    hg, xi, zs = _tc_post2(acc2, u2, dinv, gcn2_b.reshape(1, H), in_proj_W)
    xc, dt, Bm, Cm = _tc_conv(xi, conv_w.T, conv_b.reshape(1, DI), x_proj_W,
                              dt_proj_W, dt_proj_b.reshape(1, DI))
    g = _tc_scan(dt, xc, Bm, Cm, zs, hg, A_log.T, D_param.reshape(1, DI),
                 out_proj_W, ln_g.reshape(1, H), ln_b.reshape(1, H))
    ws = [critic_W, hl_W, mlt_W, p1_W, p2_W, dep_W, sel_W]
    bs = [critic_b.reshape(1, -1), hl_b.reshape(1, -1), mlt_b.reshape(1, -1),
          p1_b.reshape(1, -1), p2_b.reshape(1, -1), dep_b.reshape(1, -1),
          sel_b.reshape(1, -1)]
    outs = _tc_heads(g, ws, bs)
    return jnp.concatenate(outs, axis=-1)


# speedup vs baseline: 67.5419x; 1.1133x over previous
import functools

import jax
import jax.numpy as jnp
from jax import lax
from jax.experimental import pallas as pl
from jax.experimental.pallas import tpu as pltpu
from jax.experimental.pallas import tpu_sc as plsc

N = 10000
E = 320000
F = 128
H = 128
DI = 256
DS = 16
DC = 4
DTR = 8
NS = 10000
NLC = 8

NC = 2
NSUB = 16
NW = NC * NSUB
EB = 100
ER = E // EB
RPW = ER // NW
PH = 2
RPP = RPW // PH
RPS = N // NSUB

BLK = 1000
GN = N // BLK
CH = 200
GC = N // CH


def _sc_mesh():
    return plsc.VectorSubcoreMesh(
        core_axis_name="c", subcore_axis_name="s",
        num_cores=NC, num_subcores=NSUB)


NFC = N // 128
NTAIL = N - NFC * 128
ZB = 120
NZC = N // ZB
ZTAIL = N - NZC * ZB


def _sc_degree(dst2d):
    @functools.partial(
        pl.kernel,
        out_type=jax.ShapeDtypeStruct((NC, N, 16), jnp.float32),
        mesh=_sc_mesh(),
        scratch_types=[
            pltpu.VMEM((RPW, EB), jnp.int32),
            pltpu.VMEM((EB, 16), jnp.float32),
            pltpu.VMEM((128, 16), jnp.float32),
            pltpu.VMEM_SHARED((N, 16), jnp.float32),
            pltpu.SemaphoreType.DMA,
        ],
    )
    def k(dst_hbm, out_hbm, didx, ones_v, zbuf, hist_sh, sem):
        c = lax.axis_index("c")
        s = lax.axis_index("s")
        w = s * NC + c

        def fill_ones(i, _):
            ones_v[i, :] = jnp.ones((16,), jnp.float32)
            return 0

        lax.fori_loop(0, EB, fill_ones, 0)

        def fill_zero(i, _):
            zbuf[i, :] = jnp.zeros((16,), jnp.float32)
            return 0

        lax.fori_loop(0, 128, fill_zero, 0)

        def zero_slab(kk, _):
            i = s + NSUB * kk

            @pl.when(i < NFC)
            def _():
                pltpu.sync_copy(zbuf, hist_sh.at[pl.ds(i * 128, 128)])

            return 0

        lax.fori_loop(0, NFC // NSUB + 1, zero_slab, 0)

        @pl.when(s == 0)
        def _():
            pltpu.sync_copy(zbuf.at[pl.ds(0, NTAIL)],
                            hist_sh.at[pl.ds(NFC * 128, NTAIL)])

        plsc.subcore_barrier()

        pltpu.sync_copy(dst_hbm.at[w], didx)

        def body(j, _):
            pltpu.sync_copy(ones_v, hist_sh.at[didx.at[j]], add=True)
            return 0

        lax.fori_loop(0, RPW, body, 0)
        plsc.subcore_barrier()

        def out_slab(kk, _):
            i = s + NSUB * kk

            @pl.when(i < NFC)
            def _():
                pltpu.sync_copy(hist_sh.at[pl.ds(i * 128, 128)], zbuf)
                pltpu.sync_copy(zbuf, out_hbm.at[c, pl.ds(i * 128, 128)])

            return 0

        lax.fori_loop(0, NFC // NSUB + 1, out_slab, 0)

        @pl.when(s == 0)
        def _():
            pltpu.sync_copy(hist_sh.at[pl.ds(NFC * 128, NTAIL)],
                            zbuf.at[pl.ds(0, NTAIL)])
            pltpu.sync_copy(zbuf.at[pl.ds(0, NTAIL)],
                            out_hbm.at[c, pl.ds(NFC * 128, NTAIL)])

    return k(dst2d)


def _sc_scatter(u, src2d, dst2d):
    @functools.partial(
        pl.kernel,
        out_type=jax.ShapeDtypeStruct((NC, N, H), jnp.float32),
        mesh=_sc_mesh(),
        scratch_types=[
            pltpu.VMEM((RPP, EB), jnp.int32),
            pltpu.VMEM((RPP, EB), jnp.int32),
            pltpu.VMEM((EB, H), jnp.float32),
            pltpu.VMEM((EB, H), jnp.float32),
            pltpu.VMEM_SHARED((N, H), jnp.float32),
            pltpu.SemaphoreType.DMA,
            pltpu.SemaphoreType.DMA,
        ],
    )
    def k(u_hbm, src_hbm, dst_hbm, out_hbm, sidx, didx, rows, rows1, acc_sh,
          sem, sem1):
        c = lax.axis_index("c")
        s = lax.axis_index("s")
        w = s * NC + c

        def zero_row(i, _):
            def zero_chunk(kk, _2):
                rows[i, pl.ds(kk * 16, 16)] = jnp.zeros((16,), jnp.float32)
                return 0

            lax.fori_loop(0, H // 16, zero_chunk, 0)
            return 0

        lax.fori_loop(0, EB, zero_row, 0)

        def zero_slab(kk, _):
            i = s + NSUB * kk

            @pl.when(i < NZC)
            def _():
                pltpu.sync_copy(rows.at[pl.ds(0, ZB)],
                                acc_sh.at[pl.ds(i * ZB, ZB)])

            return 0

        lax.fori_loop(0, NZC // NSUB + 1, zero_slab, 0)

        @pl.when(s == 0)
        def _():
            pltpu.sync_copy(rows.at[pl.ds(0, ZTAIL)],
                            acc_sh.at[pl.ds(NZC * ZB, ZTAIL)])

        plsc.subcore_barrier()

        for p in range(PH):
            pltpu.sync_copy(src_hbm.at[w, p], sidx)
            pltpu.sync_copy(dst_hbm.at[w, p], didx)
            pltpu.async_copy(u_hbm.at[sidx.at[0]], rows, sem)

            def body(kk, _):
                j = 2 * kk
                pltpu.make_async_copy(u_hbm.at[sidx.at[j]], rows, sem).wait()
                pltpu.async_copy(u_hbm.at[sidx.at[j + 1]], rows1, sem1)
                pltpu.sync_copy(rows, acc_sh.at[didx.at[j]], add=True)
                pltpu.make_async_copy(u_hbm.at[sidx.at[j + 1]], rows1,
                                      sem1).wait()

                @pl.when(kk < RPP // 2 - 1)
                def _():
                    pltpu.async_copy(u_hbm.at[sidx.at[j + 2]], rows, sem)

                pltpu.sync_copy(rows1, acc_sh.at[didx.at[j + 1]], add=True)
                return 0

            lax.fori_loop(0, RPP // 2, body, 0)
        plsc.subcore_barrier()

        def out_slab(kk, _):
            i = s + NSUB * kk

            @pl.when(i < NZC)
            def _():
                pltpu.sync_copy(acc_sh.at[pl.ds(i * ZB, ZB)],
                                rows.at[pl.ds(0, ZB)])
                pltpu.sync_copy(rows.at[pl.ds(0, ZB)],
                                out_hbm.at[c, pl.ds(i * ZB, ZB)])

            return 0

        lax.fori_loop(0, NZC // NSUB + 1, out_slab, 0)

        @pl.when(s == 0)
        def _():
            pltpu.sync_copy(acc_sh.at[pl.ds(NZC * ZB, ZTAIL)],
                            rows.at[pl.ds(0, ZTAIL)])
            pltpu.sync_copy(rows.at[pl.ds(0, ZTAIL)],
                            out_hbm.at[c, pl.ds(NZC * ZB, ZTAIL)])

    return k(u, src2d, dst2d)


def _tc_pre1(x, histparts, Wp, bp, W1):
    def body(x_ref, hp_ref, wp_ref, bp_ref, w1_ref, u1_ref, dinv_ref):
        p = hp_ref[...]
        deg = p[0, :, 0:1] + p[1, :, 0:1] + 1.0
        dinv = lax.rsqrt(deg)
        h0 = jnp.dot(x_ref[...], wp_ref[...],
                     preferred_element_type=jnp.float32) + bp_ref[...]
        h0 = jnp.maximum(h0, 0.0)
        t1 = jnp.dot(h0, w1_ref[...], preferred_element_type=jnp.float32)
        u1_ref[...] = t1 * dinv
        dinv_ref[...] = dinv

    return pl.pallas_call(
        body,
        grid=(GN,),
        in_specs=[
            pl.BlockSpec((BLK, F), lambda i: (i, 0)),
            pl.BlockSpec((NC, BLK, 16), lambda i: (0, i, 0)),
            pl.BlockSpec((F, H), lambda i: (0, 0)),
            pl.BlockSpec((1, H), lambda i: (0, 0)),
            pl.BlockSpec((H, H), lambda i: (0, 0)),
        ],
        out_specs=[
            pl.BlockSpec((BLK, H), lambda i: (i, 0)),
            pl.BlockSpec((BLK, 1), lambda i: (i, 0)),
        ],
        out_shape=[
            jax.ShapeDtypeStruct((N, H), jnp.float32),
            jax.ShapeDtypeStruct((N, 1), jnp.float32),
        ],
    )(x, histparts, Wp, bp, W1)


def _tc_mid(accparts, u1, dinv, b1, W2):
    def body(p_ref, u1_ref, dinv_ref, b1_ref, w2_ref, u2_ref):
        p = p_ref[...]
        dinv = dinv_ref[...]
        g1 = dinv * (p[0] + p[1] + u1_ref[...]) + b1_ref[...]
        g1 = jnp.maximum(g1, 0.0)
        u2_ref[...] = jnp.dot(
            g1, w2_ref[...], preferred_element_type=jnp.float32) * dinv

    return pl.pallas_call(
        body,
        grid=(GN,),
        in_specs=[
            pl.BlockSpec((NC, BLK, H), lambda i: (0, i, 0)),
            pl.BlockSpec((BLK, H), lambda i: (i, 0)),
            pl.BlockSpec((BLK, 1), lambda i: (i, 0)),
            pl.BlockSpec((1, H), lambda i: (0, 0)),
            pl.BlockSpec((H, H), lambda i: (0, 0)),
        ],
        out_specs=pl.BlockSpec((BLK, H), lambda i: (i, 0)),
        out_shape=jax.ShapeDtypeStruct((N, H), jnp.float32),
    )(accparts, u1, dinv, b1, W2)


def _tc_post2(accparts, u2, dinv, b2, Win):
    def body(p_ref, u2_ref, dinv_ref, b2_ref, win_ref, hg_ref, xi_ref, zs_ref):
        p = p_ref[...]
        hg = dinv_ref[...] * (p[0] + p[1] + u2_ref[...]) + b2_ref[...]
        hg = jnp.maximum(hg, 0.0)
        hg_ref[...] = hg
        xz = jnp.dot(hg, win_ref[...], preferred_element_type=jnp.float32)
        xi_ref[...] = xz[:, :DI]
        z = xz[:, DI:]
        zs_ref[...] = z * jax.nn.sigmoid(z)

    return pl.pallas_call(
        body,
        grid=(GN,),
        in_specs=[
            pl.BlockSpec((NC, BLK, H), lambda i: (0, i, 0)),
            pl.BlockSpec((BLK, H), lambda i: (i, 0)),
            pl.BlockSpec((BLK, 1), lambda i: (i, 0)),
            pl.BlockSpec((1, H), lambda i: (0, 0)),
            pl.BlockSpec((H, 2 * DI), lambda i: (0, 0)),
        ],
        out_specs=[
            pl.BlockSpec((BLK, H), lambda i: (i, 0)),
            pl.BlockSpec((BLK, DI), lambda i: (i, 0)),
            pl.BlockSpec((BLK, DI), lambda i: (i, 0)),
        ],
        out_shape=[
            jax.ShapeDtypeStruct((N, H), jnp.float32),
            jax.ShapeDtypeStruct((N, DI), jnp.float32),
            jax.ShapeDtypeStruct((N, DI), jnp.float32),
        ],
    )(accparts, u2, dinv, b2, Win)


def _tc_conv(xi, convwT, cb, xpW, dtW, dtb):
    def body(cur_ref, prev_ref, cwT_ref, cb_ref, xpw_ref, dtw_ref, dtb_ref,
             xc_ref, dt_ref, bm_ref, cm_ref):
        pi = pl.program_id(0)
        prev_t = prev_ref[BLK - (DC - 1):BLK, :]
        prev_t = jnp.where(pi == 0, 0.0, prev_t)
        win = jnp.concatenate([prev_t, cur_ref[...]], axis=0)
        cw = cwT_ref[...]
        xc = win[DC - 1:DC - 1 + BLK] * cw[DC - 1:DC, :]
        for kk in range(DC - 1):
            xc = xc + win[kk:kk + BLK] * cw[kk:kk + 1, :]
        xc = xc + cb_ref[...]
        xc = xc * jax.nn.sigmoid(xc)
        xc_ref[...] = xc
        dbl = jnp.dot(xc, xpw_ref[...], preferred_element_type=jnp.float32)
        dtpre = jnp.dot(dbl[:, :DTR], dtw_ref[...],
                        preferred_element_type=jnp.float32) + dtb_ref[...]
        dt_ref[...] = jax.nn.softplus(dtpre)
        bm_ref[...] = dbl[:, DTR:DTR + DS]
        cm_ref[...] = dbl[:, DTR + DS:DTR + 2 * DS]

    return pl.pallas_call(
        body,
        grid=(GN,),
        in_specs=[
            pl.BlockSpec((BLK, DI), lambda i: (i, 0)),
            pl.BlockSpec((BLK, DI), lambda i: (jnp.maximum(i - 1, 0), 0)),
            pl.BlockSpec((DC, DI), lambda i: (0, 0)),
            pl.BlockSpec((1, DI), lambda i: (0, 0)),
            pl.BlockSpec((DI, DTR + 2 * DS), lambda i: (0, 0)),
            pl.BlockSpec((DTR, DI), lambda i: (0, 0)),
            pl.BlockSpec((1, DI), lambda i: (0, 0)),
        ],
        out_specs=[
            pl.BlockSpec((BLK, DI), lambda i: (i, 0)),
            pl.BlockSpec((BLK, DI), lambda i: (i, 0)),
            pl.BlockSpec((BLK, DS), lambda i: (i, 0)),
            pl.BlockSpec((BLK, DS), lambda i: (i, 0)),
        ],
        out_shape=[
            jax.ShapeDtypeStruct((N, DI), jnp.float32),
            jax.ShapeDtypeStruct((N, DI), jnp.float32),
            jax.ShapeDtypeStruct((N, DS), jnp.float32),
            jax.ShapeDtypeStruct((N, DS), jnp.float32),
        ],
    )(xi, xi, convwT, cb, xpW, dtW, dtb)


def _tc_scan(dt, xc, Bm, Cm, zs, hg, ATlog, D2, Wout, lng, lnb):
    def body(dt_ref, xc_ref, bm_ref, cm_ref, zs_ref, hg_ref, atl_ref, d_ref,
             wout_ref, lng_ref, lnb_ref, g_ref, dA_s, U_s, hst_s, h_s, gacc_s):
        pi = pl.program_id(0)
        AT = -jnp.exp(atl_ref[...])
        dt_b = dt_ref[...]
        xc_b = xc_ref[...]
        dA_s[...] = jnp.exp(dt_b[:, None, :] * AT[None])
        U_s[...] = bm_ref[...][:, :, None] * (dt_b * xc_b)[:, None, :]

        @pl.when(pi == 0)
        def _():
            h_s[...] = jnp.zeros((DS, DI), jnp.float32)
            gacc_s[...] = jnp.zeros((1, H), jnp.float32)

        def step(t, h):
            h2 = dA_s[t] * h + U_s[t]
            hst_s[t] = h2
            return h2

        h = lax.fori_loop(0, CH, step, h_s[...])
        h_s[...] = h

        Y = jnp.sum(hst_s[...] * cm_ref[...][:, :, None], axis=1)
        y = (Y + xc_b * d_ref[...]) * zs_ref[...]
        hm = jnp.dot(y, wout_ref[...], preferred_element_type=jnp.float32)
        hf = hg_ref[...] + hm
        mu = jnp.mean(hf, axis=-1, keepdims=True)
        var = jnp.mean((hf - mu) ** 2, axis=-1, keepdims=True)
        hf = (hf - mu) * lax.rsqrt(var + 1e-5) * lng_ref[...] + lnb_ref[...]
        gacc_s[...] += jnp.sum(hf, axis=0, keepdims=True)
        g_ref[...] = gacc_s[...] * (1.0 / N)

    return pl.pallas_call(
        body,
        grid=(GC,),
        in_specs=[
            pl.BlockSpec((CH, DI), lambda i: (i, 0)),
            pl.BlockSpec((CH, DI), lambda i: (i, 0)),
            pl.BlockSpec((CH, DS), lambda i: (i, 0)),
            pl.BlockSpec((CH, DS), lambda i: (i, 0)),
            pl.BlockSpec((CH, DI), lambda i: (i, 0)),
            pl.BlockSpec((CH, H), lambda i: (i, 0)),
            pl.BlockSpec((DS, DI), lambda i: (0, 0)),
            pl.BlockSpec((1, DI), lambda i: (0, 0)),
            pl.BlockSpec((DI, H), lambda i: (0, 0)),
            pl.BlockSpec((1, H), lambda i: (0, 0)),
            pl.BlockSpec((1, H), lambda i: (0, 0)),
        ],
        out_specs=pl.BlockSpec((1, H), lambda i: (0, 0)),
        out_shape=jax.ShapeDtypeStruct((1, H), jnp.float32),
        scratch_shapes=[
            pltpu.VMEM((CH, DS, DI), jnp.float32),
            pltpu.VMEM((CH, DS, DI), jnp.float32),
            pltpu.VMEM((CH, DS, DI), jnp.float32),
            pltpu.VMEM((DS, DI), jnp.float32),
            pltpu.VMEM((1, H), jnp.float32),
        ],
    )(dt, xc, Bm, Cm, zs, hg, ATlog, D2, Wout, lng, lnb)


def _tc_heads(g, ws, bs):
    nheads = len(ws)

    def body(*refs):
        g_v = refs[0][...]
        for j in range(nheads):
            w_ref = refs[1 + j]
            b_ref = refs[1 + nheads + j]
            o_ref = refs[1 + 2 * nheads + j]
            o_ref[...] = jnp.dot(
                g_v, w_ref[...], preferred_element_type=jnp.float32
            ) + b_ref[...]

    return pl.pallas_call(
        body,
        out_shape=[jax.ShapeDtypeStruct((1, w.shape[1]), jnp.float32)
                   for w in ws],
    )(g, *ws, *bs)


def kernel(node_features, edge_index, W_proj, b_proj, gcn1_W, gcn1_b, gcn2_W,
           gcn2_b, in_proj_W, conv_w, conv_b, x_proj_W, dt_proj_W, dt_proj_b,
           A_log, D_param, out_proj_W, ln_g, ln_b, critic_W, critic_b, hl_W,
           hl_b, mlt_W, mlt_b, p1_W, p1_b, p2_W, p2_b, dep_W, dep_b, sel_W,
           sel_b):
    src4d = edge_index[0].astype(jnp.int32).reshape(NW, PH, RPP, EB)
    dst4d = edge_index[1].astype(jnp.int32).reshape(NW, PH, RPP, EB)
    dst3d = dst4d.reshape(NW, RPW, EB)

    histparts = _sc_degree(dst3d)
    u1, dinv = _tc_pre1(node_features, histparts, W_proj,
                        b_proj.reshape(1, H), gcn1_W)
    acc1 = _sc_scatter(u1, src4d, dst4d)
    u2 = _tc_mid(acc1, u1, dinv, gcn1_b.reshape(1, H), gcn2_W)
    acc2 = _sc_scatter(u2, src4d, dst4d)
    hg, xi, zs = _tc_post2(acc2, u2, dinv, gcn2_b.reshape(1, H), in_proj_W)
    xc, dt, Bm, Cm = _tc_conv(xi, conv_w.T, conv_b.reshape(1, DI), x_proj_W,
                              dt_proj_W, dt_proj_b.reshape(1, DI))
    g = _tc_scan(dt, xc, Bm, Cm, zs, hg, A_log.T, D_param.reshape(1, DI),
                 out_proj_W, ln_g.reshape(1, H), ln_b.reshape(1, H))
    ws = [critic_W, hl_W, mlt_W, p1_W, p2_W, dep_W, sel_W]
    bs = [critic_b.reshape(1, -1), hl_b.reshape(1, -1), mlt_b.reshape(1, -1),
          p1_b.reshape(1, -1), p2_b.reshape(1, -1), dep_b.reshape(1, -1),
          sel_b.reshape(1, -1)]
    outs = _tc_heads(g, ws, bs)
    return jnp.concatenate(outs, axis=-1)
